# baseline (speedup 1.0000x reference)
import functools

import jax
import jax.numpy as jnp
from jax import lax
from jax.experimental import pallas as pl
from jax.experimental.pallas import tpu as pltpu
from jax.experimental.pallas import tpu_sc as plsc

B = 4096
C = 1528
P = 4000

BB = 1024
NB = B // BB

NT = 12
GFLAT = NT * C * 128

NW = 32
CHUNK = 128
PPAD = NW * CHUNK
LANES = 16


def _tc_body(x_ref, t_ref, g_ref, base_ref):
    i = pl.program_id(0)
    x = x_ref[...]
    t = t_ref[...]
    th = jnp.tanh(0.5 * x)
    s = 0.5 + 0.5 * th
    one_minus_s = 0.5 - 0.5 * th
    lg = -jnp.log(jnp.maximum(s, one_minus_s))
    bce = jnp.maximum(x, 0.0) - x * t + lg
    part = jnp.sum(bce)
    sb = s.astype(jnp.bfloat16)
    tb = one_minus_s.astype(jnp.bfloat16)
    g = lax.dot_general(sb, tb, (((1,), (1,)), ((), ())),
                        preferred_element_type=jnp.float32)

    @pl.when(i == 0)
    def _init():
        for k in range(NT - 1):
            g_ref[k] = g[:, k * 128:(k + 1) * 128]
        g_ref[NT - 1, :, 0:C - (NT - 1) * 128] = g[:, (NT - 1) * 128:]
        base_ref[0, 0] = part

    @pl.when(i > 0)
    def _acc():
        for k in range(NT - 1):
            g_ref[k] += g[:, k * 128:(k + 1) * 128]
        g_ref[NT - 1, :, 0:C - (NT - 1) * 128] += g[:, (NT - 1) * 128:]
        base_ref[0, 0] += part


def _tc_call(xt, tt):
    return pl.pallas_call(
        _tc_body,
        grid=(NB,),
        in_specs=[
            pl.BlockSpec((C, BB), lambda i: (0, i)),
            pl.BlockSpec((C, BB), lambda i: (0, i)),
        ],
        out_specs=[
            pl.BlockSpec((NT, C, 128), lambda i: (0, 0, 0)),
            pl.BlockSpec((1, 1), lambda i: (0, 0), memory_space=pltpu.SMEM),
        ],
        out_shape=[
            jax.ShapeDtypeStruct((NT, C, 128), jnp.float32),
            jax.ShapeDtypeStruct((1, 1), jnp.float32),
        ],
    )(xt, tt)


def _sc_body(g_hbm, l_hbm, r_hbm, out_hbm, l_v, r_v, idx_v, val_v, acc_v, sem):
    cid = lax.axis_index("c")
    sid = lax.axis_index("s")
    wid = sid * 2 + cid
    base = wid * CHUNK
    pltpu.sync_copy(l_hbm.at[pl.ds(base, CHUNK)], l_v)
    pltpu.sync_copy(r_hbm.at[pl.ds(base, CHUNK)], r_v)
    for j in range(CHUNK // LANES):
        sl = pl.ds(j * LANES, LANES)
        l = l_v[sl]
        r = r_v[sl]
        idx_v[sl] = ((r >> 7) * C + l) * 128 + (r & 127)
    pltpu.async_copy(g_hbm.at[idx_v], val_v, sem).wait()
    acc = jnp.zeros((LANES,), jnp.float32)
    lane = lax.iota(jnp.int32, LANES)
    for j in range(CHUNK // LANES):
        pos = base + j * LANES + lane
        v = val_v[pl.ds(j * LANES, LANES)]
        acc = acc + jnp.where(pos < P, v, 0.0)
    acc_v[...] = acc
    pltpu.sync_copy(acc_v, out_hbm.at[wid])


def _sc_call(g_flat, l_pad, r_pad):
    mesh = plsc.VectorSubcoreMesh(core_axis_name="c", subcore_axis_name="s")
    kern = functools.partial(
        pl.kernel,
        mesh=mesh,
        out_type=jax.ShapeDtypeStruct((NW, LANES), jnp.float32),
        scratch_types=[
            pltpu.VMEM((CHUNK,), jnp.int32),
            pltpu.VMEM((CHUNK,), jnp.int32),
            pltpu.VMEM((CHUNK,), jnp.int32),
            pltpu.VMEM((CHUNK,), jnp.float32),
            pltpu.VMEM((LANES,), jnp.float32),
            pltpu.SemaphoreType.DMA,
        ],
    )(_sc_body)
    return kern(g_flat, l_pad, r_pad)


def kernel(input, target, implication_filter_l, implication_filter_r):
    g3, base = _tc_call(input.T, target.T)
    l_pad = jnp.pad(implication_filter_l.astype(jnp.int32), (0, PPAD - P))
    r_pad = jnp.pad(implication_filter_r.astype(jnp.int32), (0, PPAD - P))
    partials = _sc_call(g3.reshape(-1), l_pad, r_pad)
    base_loss = base[0, 0] / (B * C)
    implication_loss = jnp.sum(partials) / B
    total = base_loss + 0.01 * implication_loss
    return (total, base_loss, implication_loss)


# speedup vs baseline: 1.0622x; 1.0622x over previous
import functools

import jax
import jax.numpy as jnp
from jax import lax
from jax.experimental import pallas as pl
from jax.experimental.pallas import tpu as pltpu
from jax.experimental.pallas import tpu_sc as plsc

B = 4096
C = 1528
P = 4000

BB = 1024
NB = B // BB

NT = 12
GFLAT = NT * C * 128

NW = 16
CHUNK = 256
LANES = 16


def _tc_body(x_ref, t_ref, g_ref, base_ref):
    i = pl.program_id(0)
    x = x_ref[...]
    t = t_ref[...]
    th = jnp.tanh(0.5 * x)
    s = 0.5 + 0.5 * th
    one_minus_s = 0.5 - 0.5 * th
    lg = -jnp.log(jnp.maximum(s, one_minus_s))
    bce = jnp.maximum(x, 0.0) - x * t + lg
    part = jnp.sum(bce)
    sb = s.astype(jnp.bfloat16)
    tb = one_minus_s.astype(jnp.bfloat16)
    g = lax.dot_general(sb, tb, (((1,), (1,)), ((), ())),
                        preferred_element_type=jnp.float32)

    @pl.when(i == 0)
    def _init():
        for k in range(NT - 1):
            g_ref[k] = g[:, k * 128:(k + 1) * 128]
        g_ref[NT - 1, :, 0:C - (NT - 1) * 128] = g[:, (NT - 1) * 128:]
        base_ref[0, 0] = part

    @pl.when(i > 0)
    def _acc():
        for k in range(NT - 1):
            g_ref[k] += g[:, k * 128:(k + 1) * 128]
        g_ref[NT - 1, :, 0:C - (NT - 1) * 128] += g[:, (NT - 1) * 128:]
        base_ref[0, 0] += part


def _tc_call(xt, tt):
    return pl.pallas_call(
        _tc_body,
        grid=(NB,),
        in_specs=[
            pl.BlockSpec((C, BB), lambda i: (0, i)),
            pl.BlockSpec((C, BB), lambda i: (0, i)),
        ],
        out_specs=[
            pl.BlockSpec((NT, C, 128), lambda i: (0, 0, 0)),
            pl.BlockSpec((1, 1), lambda i: (0, 0), memory_space=pltpu.SMEM),
        ],
        out_shape=[
            jax.ShapeDtypeStruct((NT, C, 128), jnp.float32),
            jax.ShapeDtypeStruct((1, 1), jnp.float32),
        ],
    )(xt, tt)


def _sc_body(g_hbm, l_hbm, r_hbm, out_hbm,
             l_v, r_v, idx0_v, idx1_v, val0_v, val1_v, acc_v, sem):
    wid = lax.axis_index("s")
    base = wid * CHUNK
    ngrp = CHUNK // LANES
    nvalid = P - (NW - 1) * CHUNK

    @pl.when(wid < NW - 1)
    def _full():
        pltpu.sync_copy(l_hbm.at[pl.ds(base, CHUNK)], l_v)
        pltpu.sync_copy(r_hbm.at[pl.ds(base, CHUNK)], r_v)

    @pl.when(wid == NW - 1)
    def _tail():
        pltpu.sync_copy(l_hbm.at[pl.ds(base, nvalid)], l_v.at[pl.ds(0, nvalid)])
        pltpu.sync_copy(r_hbm.at[pl.ds(base, nvalid)], r_v.at[pl.ds(0, nvalid)])

    for j in range(ngrp):
        sl = pl.ds(j * LANES, LANES)
        l = l_v[sl]
        r = r_v[sl]
        idx = ((r >> 7) * C + l) * 128 + (r & 127)
        half = pl.ds((j % (ngrp // 2)) * LANES, LANES)
        if j < ngrp // 2:
            idx0_v[half] = idx
        else:
            idx1_v[half] = idx

    @pl.when(wid == NW - 1)
    def _clamp():
        zero = jnp.zeros((LANES,), jnp.int32)
        for j in range(nvalid // LANES, ngrp):
            half = pl.ds((j % (ngrp // 2)) * LANES, LANES)
            if j < ngrp // 2:
                idx0_v[half] = zero
            else:
                idx1_v[half] = zero

    cp0 = pltpu.async_copy(g_hbm.at[idx0_v], val0_v, sem)
    cp1 = pltpu.async_copy(g_hbm.at[idx1_v], val1_v, sem)
    cp0.wait()
    cp1.wait()
    acc = jnp.zeros((LANES,), jnp.float32)
    lane = lax.iota(jnp.int32, LANES)
    for j in range(ngrp):
        pos = base + j * LANES + lane
        half = pl.ds((j % (ngrp // 2)) * LANES, LANES)
        v = val0_v[half] if j < ngrp // 2 else val1_v[half]
        acc = acc + jnp.where(pos < P, v, 0.0)
    acc_v[...] = acc
    pltpu.sync_copy(acc_v, out_hbm.at[wid])


def _sc_call(g_flat, l_idx, r_idx):
    mesh = plsc.VectorSubcoreMesh(core_axis_name="c", subcore_axis_name="s",
                                  num_cores=1)
    kern = functools.partial(
        pl.kernel,
        mesh=mesh,
        out_type=jax.ShapeDtypeStruct((NW, LANES), jnp.float32),
        scratch_types=[
            pltpu.VMEM((CHUNK,), jnp.int32),
            pltpu.VMEM((CHUNK,), jnp.int32),
            pltpu.VMEM((CHUNK // 2,), jnp.int32),
            pltpu.VMEM((CHUNK // 2,), jnp.int32),
            pltpu.VMEM((CHUNK // 2,), jnp.float32),
            pltpu.VMEM((CHUNK // 2,), jnp.float32),
            pltpu.VMEM((LANES,), jnp.float32),
            pltpu.SemaphoreType.DMA,
        ],
    )(_sc_body)
    return kern(g_flat, l_idx, r_idx)


def kernel(input, target, implication_filter_l, implication_filter_r):
    g3, base = _tc_call(input.T, target.T)
    partials = _sc_call(g3.reshape(-1),
                        implication_filter_l.astype(jnp.int32),
                        implication_filter_r.astype(jnp.int32))
    base_loss = base[0, 0] / (B * C)
    implication_loss = jnp.sum(partials) / B
    total = base_loss + 0.01 * implication_loss
    return (total, base_loss, implication_loss)
